# Initial kernel scaffold; baseline (speedup 1.0000x reference)
import jax
import jax.numpy as jnp
from jax.experimental import pallas as pl


def kernel(x, src, tgt, Mtgt, W0, b0, gc1W, gc1b, gn1w, gn1b, gc2W, gc2b, gn2w, gn2b, Wl, bl):
    raise NotImplementedError("write your pallas kernel here")


# speedup vs baseline: 2.2057x; 2.2057x over previous
import functools

import jax
import jax.numpy as jnp
import numpy as np
from jax import lax
from jax.experimental import pallas as pl
from jax.experimental.pallas import tpu as pltpu
from jax.experimental.pallas import tpu_sc as plsc

N = 10000
E = 160000
NFEAT = 256
NHID = 256
NCLASS = 40
GROUPS = 32
DT = 0.5

BLK = 1000
NTILES = 16
CHUNK = 128
NCH = 80
EPAD = NTILES * NCH * CHUNK
TILE_STRIDE = 632
ACC_ROWS = NTILES * TILE_STRIDE


def _make_agg(D2, nphases):
    mesh = plsc.VectorSubcoreMesh(core_axis_name="c", subcore_axis_name="s")
    nvec = D2 // 16
    nsup = 2 * nphases
    f32 = jnp.float32

    def body(*refs):
        sups = refs[:nsup]
        srcT, tgtT, mT = refs[nsup:nsup + 3]
        outs = refs[nsup + 3:nsup + 3 + nsup]
        src_v, tgt_v, m_v, bufs, acc, sem0, sem1 = refs[nsup + 3 + nsup:]

        cid = lax.axis_index("c")
        sid = lax.axis_index("s")

        pltpu.sync_copy(srcT.at[sid], src_v)
        pltpu.sync_copy(tgtT.at[sid], tgt_v)
        pltpu.sync_copy(mT.at[sid], m_v)

        base = sid * TILE_STRIDE
        zv = jnp.zeros((16,), f32)

        def run(sup, out, nrows_drain):
            def zrow(i, carry):
                for q in range(nvec):
                    bufs[0, i, pl.ds(q * 16, 16)] = zv
                return carry
            lax.fori_loop(0, CHUNK, zrow, 0)
            off = 0
            while off < TILE_STRIDE:
                nr = min(CHUNK, TILE_STRIDE - off)
                pltpu.sync_copy(bufs.at[0, pl.ds(0, nr)],
                                acc.at[pl.ds(base + off, nr)])
                off += nr
            plsc.subcore_barrier()

            sems = (sem0, sem1)

            def start(j, bank):
                pltpu.async_copy(sup.at[src_v.at[j]], bufs.at[bank],
                                 sems[bank])

            def wait(j, bank):
                pltpu.make_async_copy(sup.at[src_v.at[j]], bufs.at[bank],
                                      sems[bank]).wait()

            def scale(j, bank):
                jbase = j * CHUNK

                def sgroup(g, carry):
                    v16 = m_v[pl.ds(jbase + g * 16, 16)]
                    for l in range(16):
                        m = jnp.broadcast_to(v16[l], (16,))
                        i = g * 16 + l
                        for q in range(nvec):
                            v = bufs[bank, i, pl.ds(q * 16, 16)]
                            bufs[bank, i, pl.ds(q * 16, 16)] = v * m
                    return carry
                lax.fori_loop(0, CHUNK // 16, sgroup, 0)

            def scatter(j, bank):
                pltpu.sync_copy(bufs.at[bank], acc.at[tgt_v.at[j]], add=True)

            start(0, 0)

            def step2(jj, carry):
                j = 2 * jj
                start(j + 1, 1)
                wait(j, 0)
                scale(j, 0)
                scatter(j, 0)

                @pl.when(j + 2 < NCH)
                def _():
                    start(j + 2, 0)
                wait(j + 1, 1)
                scale(j + 1, 1)
                scatter(j + 1, 1)
                return carry
            lax.fori_loop(0, NCH // 2, step2, 0)

            plsc.subcore_barrier()
            o = 0
            while o < nrows_drain:
                nr = min(CHUNK, nrows_drain - o)
                pltpu.sync_copy(acc.at[pl.ds(base + o, nr)],
                                bufs.at[0, pl.ds(0, nr)])
                pltpu.sync_copy(bufs.at[0, pl.ds(0, nr)],
                                out.at[pl.ds(base + o, nr)])
                o += nr

        last_rows = N - (NTILES - 1) * TILE_STRIDE

        for c in range(2):
            @pl.when(cid == c)
            def _():
                for p in range(nphases):
                    i = nphases * c + p

                    @pl.when(sid < NTILES - 1)
                    def _():
                        run(sups[i], outs[i], TILE_STRIDE)

                    @pl.when(sid == NTILES - 1)
                    def _():
                        run(sups[i], outs[i], last_rows)

    sds = jax.ShapeDtypeStruct((N, D2), f32)
    return pl.kernel(
        body,
        out_type=(sds,) * nsup,
        mesh=mesh,
        scratch_types=[
            pltpu.VMEM((NCH, CHUNK), jnp.int32),
            pltpu.VMEM((NCH, CHUNK), jnp.int32),
            pltpu.VMEM((NCH * CHUNK,), f32),
            pltpu.VMEM((2, CHUNK, D2), f32),
            pltpu.VMEM_SHARED((ACC_ROWS, D2), f32),
            pltpu.SemaphoreType.DMA,
            pltpu.SemaphoreType.DMA,
        ],
        compiler_params=pltpu.CompilerParams(use_tc_tiling_on_sc=False),
    )


_agg_main = _make_agg(64, 2)
_agg_cls = _make_agg(32, 1)


def _gn(z, A8, w, b):
    m = jnp.dot(z, A8, preferred_element_type=jnp.float32)
    q = jnp.dot(z * z, A8, preferred_element_type=jnp.float32)
    inv = lax.rsqrt(q - m * m + 1e-5)
    return (z - m) * inv * w + b


def _row_spec(w):
    return pl.BlockSpec((BLK, w), lambda i: (i, 0))


def _full_spec(shape):
    nd = len(shape)
    return pl.BlockSpec(shape, lambda i: (0,) * nd)


def _tc_call(body, in_specs, out_specs, out_shapes):
    return pl.pallas_call(
        body,
        grid=(N // BLK,),
        in_specs=in_specs,
        out_specs=out_specs,
        out_shape=out_shapes,
    )


def _mm_split(s, outs, w):
    for i, o in enumerate(outs):
        o[...] = s[:, i * w:(i + 1) * w]


def _cat(gs):
    return jnp.concatenate([g[...] for g in gs], axis=1)


def _mmA_body(y, V, bias, *outs):
    s = jnp.dot(y[...], V[...], preferred_element_type=jnp.float32) + bias[...]
    _mm_split(s, outs, 64)


def _mmB0_body(gA, gB, gC, gD, V, bias, *outs):
    y = jnp.maximum(_cat((gA, gB, gC, gD)), 0.0)
    outs[-1][...] = y
    s = jnp.dot(y, V[...], preferred_element_type=jnp.float32) + bias[...]
    _mm_split(s, outs[:-1], 64)


def _mmB_body(gA, gB, gC, gD, w1, b1, A8, V, bias, *outs):
    z = jnp.maximum(_cat((gA, gB, gC, gD)), 0.0)
    h = _gn(z, A8[...], w1[...], b1[...])
    s = jnp.dot(h, V[...], preferred_element_type=jnp.float32) + bias[...]
    _mm_split(s, outs, 64)


def _mmC_body(gA, gB, gC, gD, y, acc, w2, b2, A8, V, bias, *outs,
              c_u, w_acc, step_end, w_out):
    z = jnp.maximum(_cat((gA, gB, gC, gD)), 0.0)
    k = _gn(z, A8[...], w2[...], b2[...])
    nacc = acc[...] + w_acc * k
    outs[-1][...] = nacc
    u = nacc if step_end else y[...] + c_u * k
    s = jnp.dot(u, V[...], preferred_element_type=jnp.float32) + bias[...]
    _mm_split(s, outs[:-1], w_out)


def _ls_body(gA, gB, out):
    z = jnp.concatenate([gA[...], gB[...]], axis=1)
    lane = lax.broadcasted_iota(jnp.int32, z.shape, 1)
    valid = lane < NCLASS
    zm = jnp.where(valid, z, -jnp.inf)
    mx = jnp.max(zm, axis=1, keepdims=True)
    e = jnp.where(valid, jnp.exp(z - mx), 0.0)
    lse = jnp.log(jnp.sum(e, axis=1, keepdims=True)) + mx
    out[...] = (z - lse)[:, :NCLASS]


def _sds(shape):
    return jax.ShapeDtypeStruct(shape, jnp.float32)


_q4_specs = tuple(_row_spec(64) for _ in range(4))
_q4_shapes = tuple(_sds((N, 64)) for _ in range(4))

_mmA = _tc_call(
    _mmA_body,
    [_row_spec(NFEAT), _full_spec((NFEAT, NHID)), _full_spec((1, NHID))],
    _q4_specs,
    _q4_shapes,
)

_mmB0 = _tc_call(
    _mmB0_body,
    list(_q4_specs) + [_full_spec((NHID, NHID)), _full_spec((1, NHID))],
    _q4_specs + (_row_spec(NHID),),
    _q4_shapes + (_sds((N, NHID)),),
)

_mmB = _tc_call(
    _mmB_body,
    list(_q4_specs) + [_full_spec((1, NHID)), _full_spec((1, NHID)),
                       _full_spec((NHID, NHID)), _full_spec((NHID, NHID)),
                       _full_spec((1, NHID))],
    _q4_specs,
    _q4_shapes,
)


def _make_mmC(c_u, w_acc, step_end, nout, w_out):
    body = functools.partial(_mmC_body, c_u=c_u, w_acc=w_acc,
                             step_end=step_end, w_out=w_out)
    osp = tuple(_row_spec(w_out) for _ in range(nout)) + (_row_spec(NHID),)
    osh = tuple(_sds((N, w_out)) for _ in range(nout)) + (_sds((N, NHID)),)
    return _tc_call(
        body,
        list(_q4_specs) + [_row_spec(NHID), _row_spec(NHID),
                           _full_spec((1, NHID)), _full_spec((1, NHID)),
                           _full_spec((NHID, NHID)),
                           _full_spec((NHID, nout * w_out)),
                           _full_spec((1, nout * w_out))],
        osp,
        osh,
    )


_ls = _tc_call(
    _ls_body,
    [_row_spec(32), _row_spec(32)],
    _row_spec(NCLASS),
    _sds((N, NCLASS)),
)

_A8_NP = np.kron(np.eye(GROUPS), np.full((8, 8), 0.125)).astype(np.float32)


def kernel(x, src, tgt, Mtgt, W0, b0, gc1W, gc1b, gn1w, gn1b,
           gc2W, gc2b, gn2w, gn2b, Wl, bl):
    pad = EPAD - E
    srcT = jnp.pad(src, (0, pad)).reshape(NTILES, NCH, CHUNK)
    tgtT = jnp.pad(tgt, (0, pad)).reshape(NTILES, NCH, CHUNK)
    mT = jnp.pad(Mtgt[:, 0], (0, pad)).reshape(NTILES, NCH * CHUNK)

    _A8 = jnp.asarray(_A8_NP)
    V1, r1 = gc1W[1:], gc1W[0]
    V2, r2 = gc2W[1:], gc2W[0]

    def b1(t):
        return (gc1b + t * r1).reshape(1, NHID)

    def b2(t):
        return (gc2b + t * r2).reshape(1, NHID)

    WlP = jnp.pad(Wl, ((0, 0), (0, 64 - NCLASS)))
    blP = jnp.pad(bl, (0, 64 - NCLASS)).reshape(1, 64)
    gw1 = gn1w.reshape(1, NHID)
    gb1 = gn1b.reshape(1, NHID)
    gw2 = gn2w.reshape(1, NHID)
    gb2 = gn2b.reshape(1, NHID)

    def agg(s4):
        return _agg_main(*s4, srcT, tgtT, mT)

    s4 = _mmA(x, W0, b0.reshape(1, NHID))
    g4 = agg(s4)
    *s4, y = _mmB0(*g4, V1, b1(0.0))
    acc = y

    stage_t = [0.0, 0.25, 0.25, 0.5, 0.5, 0.75, 0.75, 1.0]
    cu = [DT / 2, DT / 2, DT, 0.0]
    wa = [DT / 6, DT / 3, DT / 3, DT / 6]
    for j in range(8):
        pos = j % 4
        g4 = agg(s4)
        s4 = _mmB(*g4, gw1, gb1, _A8, V2, b2(stage_t[j]))
        g4 = agg(s4)
        last = j == 7
        step_end = pos == 3
        mmC = _make_mmC(cu[pos], wa[pos], step_end,
                        2 if last else 4, 32 if last else 64)
        outs = mmC(*g4, y, acc,  gw2, gb2, _A8,
                   WlP if last else V1,
                   blP if last else b1(stage_t[j + 1] if not last else 0.0))
        *s4, newst = outs
        if step_end:
            y = newst
            acc = newst
        else:
            acc = newst

    gA, gB = _agg_cls(*s4, srcT, tgtT, mT)
    return _ls(gA, gB)


# speedup vs baseline: 3.1419x; 1.4245x over previous
import functools

import jax
import jax.numpy as jnp
import numpy as np
from jax import lax
from jax.experimental import pallas as pl
from jax.experimental.pallas import tpu as pltpu
from jax.experimental.pallas import tpu_sc as plsc

N = 10000
E = 160000
NFEAT = 256
NHID = 256
NCLASS = 40
GROUPS = 32
DT = 0.5

BLK = 1000
NTILES = 16
CHUNK = 128
NCH = 80
EPAD = NTILES * NCH * CHUNK
TILE_STRIDE = 632
ACC_ROWS = NTILES * TILE_STRIDE


def _make_agg(D2, nphases):
    mesh = plsc.VectorSubcoreMesh(core_axis_name="c", subcore_axis_name="s")
    nvec = D2 // 16
    nsup = 2 * nphases
    f32 = jnp.float32

    def body(*refs):
        sups = refs[:nsup]
        srcT, tgtT, mT = refs[nsup:nsup + 3]
        outs = refs[nsup + 3:nsup + 3 + nsup]
        (src_v, tgt_v, m_v, bin_, bout, acc,
         gsem0, gsem1, ssem0, ssem1) = refs[nsup + 3 + nsup:]

        cid = lax.axis_index("c")
        sid = lax.axis_index("s")

        pltpu.sync_copy(srcT.at[sid], src_v)
        pltpu.sync_copy(tgtT.at[sid], tgt_v)
        pltpu.sync_copy(mT.at[sid], m_v)

        base = sid * TILE_STRIDE
        zv = jnp.zeros((16,), f32)

        gsems = (gsem0, gsem1)
        ssems = (ssem0, ssem1)

        def run(sup, out, nrows_drain):
            def zrow(i, carry):
                for q in range(nvec):
                    bout[0, i, pl.ds(q * 16, 16)] = zv
                return carry
            lax.fori_loop(0, CHUNK, zrow, 0)
            off = 0
            while off < TILE_STRIDE:
                nr = min(CHUNK, TILE_STRIDE - off)
                pltpu.sync_copy(bout.at[0, pl.ds(0, nr)],
                                acc.at[pl.ds(base + off, nr)])
                off += nr
            plsc.subcore_barrier()

            def gstart(j, bank):
                pltpu.async_copy(sup.at[src_v.at[j]], bin_.at[bank],
                                 gsems[bank])

            def gwait(j, bank):
                pltpu.make_async_copy(sup.at[src_v.at[j]], bin_.at[bank],
                                      gsems[bank]).wait()

            def scale(j, bank):
                jbase = j * CHUNK

                def sgroup(g, carry):
                    v16 = m_v[pl.ds(jbase + g * 16, 16)]
                    for l in range(16):
                        m = jnp.broadcast_to(v16[l], (16,))
                        i = g * 16 + l
                        for q in range(nvec):
                            v = bin_[bank, i, pl.ds(q * 16, 16)]
                            bout[bank, i, pl.ds(q * 16, 16)] = v * m
                    return carry
                lax.fori_loop(0, CHUNK // 16, sgroup, 0)

            def sstart(j, bank):
                pltpu.async_copy(bout.at[bank], acc.at[tgt_v.at[j]],
                                 ssems[bank], add=True)

            def swait(j, bank):
                pltpu.make_async_copy(bout.at[bank], acc.at[tgt_v.at[j]],
                                      ssems[bank]).wait()

            gstart(0, 0)
            gstart(1, 1)

            def step2(jj, carry):
                j = 2 * jj
                for bank in range(2):
                    jc = j + bank
                    gwait(jc, bank)

                    @pl.when(jc >= 2)
                    def _():
                        swait(jc - 2, bank)
                    scale(jc, bank)
                    sstart(jc, bank)

                    @pl.when(jc + 2 < NCH)
                    def _():
                        gstart(jc + 2, bank)
                return carry
            lax.fori_loop(0, NCH // 2, step2, 0)
            swait(NCH - 2, 0)
            swait(NCH - 1, 1)

            plsc.subcore_barrier()
            o = 0
            while o < nrows_drain:
                nr = min(CHUNK, nrows_drain - o)
                pltpu.sync_copy(acc.at[pl.ds(base + o, nr)],
                                out.at[pl.ds(base + o, nr)])
                o += nr

        last_rows = N - (NTILES - 1) * TILE_STRIDE

        for c in range(2):
            @pl.when(cid == c)
            def _():
                for p in range(nphases):
                    i = nphases * c + p

                    @pl.when(sid < NTILES - 1)
                    def _():
                        run(sups[i], outs[i], TILE_STRIDE)

                    @pl.when(sid == NTILES - 1)
                    def _():
                        run(sups[i], outs[i], last_rows)

    sds = jax.ShapeDtypeStruct((N, D2), f32)
    return pl.kernel(
        body,
        out_type=(sds,) * nsup,
        mesh=mesh,
        scratch_types=[
            pltpu.VMEM((NCH, CHUNK), jnp.int32),
            pltpu.VMEM((NCH, CHUNK), jnp.int32),
            pltpu.VMEM((NCH * CHUNK,), f32),
            pltpu.VMEM((2, CHUNK, D2), f32),
            pltpu.VMEM((2, CHUNK, D2), f32),
            pltpu.VMEM_SHARED((ACC_ROWS, D2), f32),
            pltpu.SemaphoreType.DMA,
            pltpu.SemaphoreType.DMA,
            pltpu.SemaphoreType.DMA,
            pltpu.SemaphoreType.DMA,
        ],
        compiler_params=pltpu.CompilerParams(use_tc_tiling_on_sc=False),
    )


_agg_main = _make_agg(64, 2)
_agg_cls = _make_agg(32, 1)


def _gn(z, A8, w, b):
    m = jnp.dot(z, A8, preferred_element_type=jnp.float32)
    q = jnp.dot(z * z, A8, preferred_element_type=jnp.float32)
    inv = lax.rsqrt(q - m * m + 1e-5)
    return (z - m) * inv * w + b


def _row_spec(w):
    return pl.BlockSpec((BLK, w), lambda i: (i, 0))


def _full_spec(shape):
    nd = len(shape)
    return pl.BlockSpec(shape, lambda i: (0,) * nd)


def _tc_call(body, in_specs, out_specs, out_shapes):
    return pl.pallas_call(
        body,
        grid=(N // BLK,),
        in_specs=in_specs,
        out_specs=out_specs,
        out_shape=out_shapes,
    )


def _mm_split(s, outs, w):
    for i, o in enumerate(outs):
        o[...] = s[:, i * w:(i + 1) * w]


def _cat(gs):
    return jnp.concatenate([g[...] for g in gs], axis=1)


def _mmA_body(y, V, bias, *outs):
    s = jnp.dot(y[...], V[...], preferred_element_type=jnp.float32) + bias[...]
    _mm_split(s, outs, 64)


def _mmB0_body(gA, gB, gC, gD, V, bias, *outs):
    y = jnp.maximum(_cat((gA, gB, gC, gD)), 0.0)
    outs[-1][...] = y
    s = jnp.dot(y, V[...], preferred_element_type=jnp.float32) + bias[...]
    _mm_split(s, outs[:-1], 64)


def _mmB_body(gA, gB, gC, gD, w1, b1, A8, V, bias, *outs):
    z = jnp.maximum(_cat((gA, gB, gC, gD)), 0.0)
    h = _gn(z, A8[...], w1[...], b1[...])
    s = jnp.dot(h, V[...], preferred_element_type=jnp.float32) + bias[...]
    _mm_split(s, outs, 64)


def _mmC_body(gA, gB, gC, gD, y, acc, w2, b2, A8, V, bias, *outs,
              c_u, w_acc, step_end, w_out):
    z = jnp.maximum(_cat((gA, gB, gC, gD)), 0.0)
    k = _gn(z, A8[...], w2[...], b2[...])
    nacc = acc[...] + w_acc * k
    outs[-1][...] = nacc
    u = nacc if step_end else y[...] + c_u * k
    s = jnp.dot(u, V[...], preferred_element_type=jnp.float32) + bias[...]
    _mm_split(s, outs[:-1], w_out)


def _ls_body(gA, gB, out):
    z = jnp.concatenate([gA[...], gB[...]], axis=1)
    lane = lax.broadcasted_iota(jnp.int32, z.shape, 1)
    valid = lane < NCLASS
    zm = jnp.where(valid, z, -jnp.inf)
    mx = jnp.max(zm, axis=1, keepdims=True)
    e = jnp.where(valid, jnp.exp(z - mx), 0.0)
    lse = jnp.log(jnp.sum(e, axis=1, keepdims=True)) + mx
    out[...] = (z - lse)[:, :NCLASS]


def _sds(shape):
    return jax.ShapeDtypeStruct(shape, jnp.float32)


_q4_specs = tuple(_row_spec(64) for _ in range(4))
_q4_shapes = tuple(_sds((N, 64)) for _ in range(4))

_mmA = _tc_call(
    _mmA_body,
    [_row_spec(NFEAT), _full_spec((NFEAT, NHID)), _full_spec((1, NHID))],
    _q4_specs,
    _q4_shapes,
)

_mmB0 = _tc_call(
    _mmB0_body,
    list(_q4_specs) + [_full_spec((NHID, NHID)), _full_spec((1, NHID))],
    _q4_specs + (_row_spec(NHID),),
    _q4_shapes + (_sds((N, NHID)),),
)

_mmB = _tc_call(
    _mmB_body,
    list(_q4_specs) + [_full_spec((1, NHID)), _full_spec((1, NHID)),
                       _full_spec((NHID, NHID)), _full_spec((NHID, NHID)),
                       _full_spec((1, NHID))],
    _q4_specs,
    _q4_shapes,
)


def _make_mmC(c_u, w_acc, step_end, nout, w_out):
    body = functools.partial(_mmC_body, c_u=c_u, w_acc=w_acc,
                             step_end=step_end, w_out=w_out)
    osp = tuple(_row_spec(w_out) for _ in range(nout)) + (_row_spec(NHID),)
    osh = tuple(_sds((N, w_out)) for _ in range(nout)) + (_sds((N, NHID)),)
    return _tc_call(
        body,
        list(_q4_specs) + [_row_spec(NHID), _row_spec(NHID),
                           _full_spec((1, NHID)), _full_spec((1, NHID)),
                           _full_spec((NHID, NHID)),
                           _full_spec((NHID, nout * w_out)),
                           _full_spec((1, nout * w_out))],
        osp,
        osh,
    )


_ls = _tc_call(
    _ls_body,
    [_row_spec(32), _row_spec(32)],
    _row_spec(NCLASS),
    _sds((N, NCLASS)),
)

_A8_NP = np.kron(np.eye(GROUPS), np.full((8, 8), 0.125)).astype(np.float32)


def kernel(x, src, tgt, Mtgt, W0, b0, gc1W, gc1b, gn1w, gn1b,
           gc2W, gc2b, gn2w, gn2b, Wl, bl):
    pad = EPAD - E
    srcT = jnp.pad(src, (0, pad)).reshape(NTILES, NCH, CHUNK)
    tgtT = jnp.pad(tgt, (0, pad)).reshape(NTILES, NCH, CHUNK)
    mT = jnp.pad(Mtgt[:, 0], (0, pad)).reshape(NTILES, NCH * CHUNK)

    _A8 = jnp.asarray(_A8_NP)
    V1, r1 = gc1W[1:], gc1W[0]
    V2, r2 = gc2W[1:], gc2W[0]

    def b1(t):
        return (gc1b + t * r1).reshape(1, NHID)

    def b2(t):
        return (gc2b + t * r2).reshape(1, NHID)

    WlP = jnp.pad(Wl, ((0, 0), (0, 64 - NCLASS)))
    blP = jnp.pad(bl, (0, 64 - NCLASS)).reshape(1, 64)
    gw1 = gn1w.reshape(1, NHID)
    gb1 = gn1b.reshape(1, NHID)
    gw2 = gn2w.reshape(1, NHID)
    gb2 = gn2b.reshape(1, NHID)

    def agg(s4):
        return _agg_main(*s4, srcT, tgtT, mT)

    s4 = _mmA(x, W0, b0.reshape(1, NHID))
    g4 = agg(s4)
    *s4, y = _mmB0(*g4, V1, b1(0.0))
    acc = y

    stage_t = [0.0, 0.25, 0.25, 0.5, 0.5, 0.75, 0.75, 1.0]
    cu = [DT / 2, DT / 2, DT, 0.0]
    wa = [DT / 6, DT / 3, DT / 3, DT / 6]
    for j in range(8):
        pos = j % 4
        g4 = agg(s4)
        s4 = _mmB(*g4, gw1, gb1, _A8, V2, b2(stage_t[j]))
        g4 = agg(s4)
        last = j == 7
        step_end = pos == 3
        mmC = _make_mmC(cu[pos], wa[pos], step_end,
                        2 if last else 4, 32 if last else 64)
        outs = mmC(*g4, y, acc,  gw2, gb2, _A8,
                   WlP if last else V1,
                   blP if last else b1(stage_t[j + 1] if not last else 0.0))
        *s4, newst = outs
        if step_end:
            y = newst
            acc = newst
        else:
            acc = newst

    gA, gB = _agg_cls(*s4, srcT, tgtT, mT)
    return _ls(gA, gB)


# speedup vs baseline: 3.3698x; 1.0725x over previous
import functools

import jax
import jax.numpy as jnp
import numpy as np
from jax import lax
from jax.experimental import pallas as pl
from jax.experimental.pallas import tpu as pltpu
from jax.experimental.pallas import tpu_sc as plsc

N = 10000
E = 160000
NFEAT = 256
NHID = 256
NCLASS = 40
GROUPS = 32
DT = 0.5

BLK = 1000
NTILES = 16
CHUNK = 128
NCH = 80
EPAD = NTILES * NCH * CHUNK
TILE_STRIDE = 632
_SCATTER = True
_SCALE = True
ACC_ROWS = NTILES * TILE_STRIDE


def _make_agg(D2, nphases, acc_rows=ACC_ROWS, accio=True):
    mesh = plsc.VectorSubcoreMesh(core_axis_name="c", subcore_axis_name="s")
    nvec = D2 // 16
    nsup = 2 * nphases
    f32 = jnp.float32

    def body(*refs):
        sups = refs[:nsup]
        srcT, tgtT, mT = refs[nsup:nsup + 3]
        outs = refs[nsup + 3:nsup + 3 + nsup]
        (src_v, tgt_v, m_v, bin_, bout, acc,
         gsem0, gsem1, ssem0, ssem1) = refs[nsup + 3 + nsup:]

        cid = lax.axis_index("c")
        sid = lax.axis_index("s")

        pltpu.sync_copy(srcT.at[sid], src_v)
        pltpu.sync_copy(tgtT.at[sid], tgt_v)
        pltpu.sync_copy(mT.at[sid], m_v)

        base = sid * TILE_STRIDE
        zv = jnp.zeros((16,), f32)

        gsems = (gsem0, gsem1)
        ssems = (ssem0, ssem1)

        def run(sup, out, nrows_drain):
            def zrow(i, carry):
                for q in range(nvec):
                    bout[0, i, pl.ds(q * 16, 16)] = zv
                return carry
            lax.fori_loop(0, CHUNK, zrow, 0)
            if accio:
                off = 0
                while off < TILE_STRIDE:
                    nr = min(CHUNK, TILE_STRIDE - off)
                    pltpu.sync_copy(bout.at[0, pl.ds(0, nr)],
                                    acc.at[pl.ds(base + off, nr)])
                    off += nr
            plsc.subcore_barrier()

            def gstart(j, bank):
                pltpu.async_copy(sup.at[src_v.at[j]], bin_.at[bank],
                                 gsems[bank])

            def gwait(j, bank):
                pltpu.make_async_copy(sup.at[src_v.at[j]], bin_.at[bank],
                                      gsems[bank]).wait()

            def scale(j, bank):
                jbase = j * CHUNK
                hmask = jnp.full((16,), -65536, jnp.int32)

                def sgroup(g, carry):
                    v16 = m_v[pl.ds(jbase + g * 16, 16)]
                    for l in range(16):
                        m = jnp.broadcast_to(v16[l], (16,))
                        i = g * 16 + l
                        for w in range(D2 // 32):
                            v = bin_[bank, i, pl.ds(w * 16, 16)]
                            lo = lax.bitcast_convert_type(v << 16,
                                                          jnp.float32)
                            hi = lax.bitcast_convert_type(v & hmask,
                                                          jnp.float32)
                            bout[bank, i, pl.ds(w * 32, 16)] = lo * m
                            bout[bank, i, pl.ds(w * 32 + 16, 16)] = hi * m
                    return carry
                if _SCALE:
                    lax.fori_loop(0, CHUNK // 16, sgroup, 0)

            def sstart(j, bank):
                pltpu.async_copy(bout.at[bank], acc.at[tgt_v.at[j]],
                                 ssems[bank], add=True)

            def swait(j, bank):
                pltpu.make_async_copy(bout.at[bank], acc.at[tgt_v.at[j]],
                                      ssems[bank]).wait()

            gstart(0, 0)
            gstart(1, 1)

            def step2(jj, carry):
                j = 2 * jj
                for bank in range(2):
                    jc = j + bank
                    gwait(jc, bank)

                    if _SCATTER:
                        @pl.when(jc >= 2)
                        def _():
                            swait(jc - 2, bank)
                    scale(jc, bank)
                    if _SCATTER:
                        sstart(jc, bank)

                    @pl.when(jc + 2 < NCH)
                    def _():
                        gstart(jc + 2, bank)
                return carry
            lax.fori_loop(0, NCH // 2, step2, 0)
            if _SCATTER:
                swait(NCH - 2, 0)
                swait(NCH - 1, 1)

            plsc.subcore_barrier()
            if accio:
                o = 0
                while o < nrows_drain:
                    nr = min(CHUNK, nrows_drain - o)
                    pltpu.sync_copy(acc.at[pl.ds(base + o, nr)],
                                    out.at[pl.ds(base + o, nr)])
                    o += nr

        last_rows = N - (NTILES - 1) * TILE_STRIDE

        for c in range(2):
            @pl.when(cid == c)
            def _():
                for p in range(nphases):
                    i = nphases * c + p

                    @pl.when(sid < NTILES - 1)
                    def _():
                        run(sups[i], outs[i], TILE_STRIDE)

                    @pl.when(sid == NTILES - 1)
                    def _():
                        run(sups[i], outs[i], last_rows)

    sds = jax.ShapeDtypeStruct((N, D2), f32)
    return pl.kernel(
        body,
        out_type=(sds,) * nsup,
        mesh=mesh,
        scratch_types=[
            pltpu.VMEM((NCH, CHUNK), jnp.int32),
            pltpu.VMEM((NCH, CHUNK), jnp.int32),
            pltpu.VMEM((NCH * CHUNK,), f32),
            pltpu.VMEM((2, CHUNK, D2 // 2), jnp.int32),
            pltpu.VMEM((2, CHUNK, D2), f32),
            pltpu.VMEM_SHARED((acc_rows, D2), f32),
            pltpu.SemaphoreType.DMA,
            pltpu.SemaphoreType.DMA,
            pltpu.SemaphoreType.DMA,
            pltpu.SemaphoreType.DMA,
        ],
        compiler_params=pltpu.CompilerParams(
            use_tc_tiling_on_sc=False,
            internal_scratch_in_bytes=256 * 1024,
        ),
    )


_agg_main = _make_agg(64, 2)
_agg_cls = _make_agg(32, 1)


def _gn(z, A8, w, b):
    m = jnp.dot(z, A8, preferred_element_type=jnp.float32)
    q = jnp.dot(z * z, A8, preferred_element_type=jnp.float32)
    inv = lax.rsqrt(q - m * m + 1e-5)
    return (z - m) * inv * w + b


def _row_spec(w):
    return pl.BlockSpec((BLK, w), lambda i: (i, 0))


def _full_spec(shape):
    nd = len(shape)
    return pl.BlockSpec(shape, lambda i: (0,) * nd)


def _tc_call(body, in_specs, out_specs, out_shapes):
    return pl.pallas_call(
        body,
        grid=(N // BLK,),
        in_specs=in_specs,
        out_specs=out_specs,
        out_shape=out_shapes,
    )


def _pack(q):
    blocks = []
    for b in range(q.shape[1] // 32):
        lo = q[:, 32 * b:32 * b + 16]
        hi = q[:, 32 * b + 16:32 * b + 32]
        lo16 = lax.bitcast_convert_type(lo.astype(jnp.bfloat16), jnp.uint16)
        hi16 = lax.bitcast_convert_type(hi.astype(jnp.bfloat16), jnp.uint16)
        w = lo16.astype(jnp.uint32) | (hi16.astype(jnp.uint32) << 16)
        blocks.append(lax.bitcast_convert_type(w, jnp.int32))
    return jnp.concatenate(blocks, axis=1) if len(blocks) > 1 else blocks[0]


def _mm_split(s, outs, w):
    for i, o in enumerate(outs):
        o[...] = _pack(s[:, i * w:(i + 1) * w])


def _cat(gs):
    return jnp.concatenate([g[...] for g in gs], axis=1)


def _mmA_body(y, V, bias, *outs):
    s = jnp.dot(y[...], V[...], preferred_element_type=jnp.float32) + bias[...]
    _mm_split(s, outs, 64)


def _mmB0_body(gA, gB, gC, gD, V, bias, *outs):
    y = jnp.maximum(_cat((gA, gB, gC, gD)), 0.0)
    outs[-1][...] = y
    s = jnp.dot(y, V[...], preferred_element_type=jnp.float32) + bias[...]
    _mm_split(s, outs[:-1], 64)


def _mmB_body(gA, gB, gC, gD, w1, b1, A8, V, bias, *outs):
    z = jnp.maximum(_cat((gA, gB, gC, gD)), 0.0)
    h = _gn(z, A8[...], w1[...], b1[...])
    s = jnp.dot(h, V[...], preferred_element_type=jnp.float32) + bias[...]
    _mm_split(s, outs, 64)


def _mmC_body(gA, gB, gC, gD, y, acc, w2, b2, A8, V, bias, *outs,
              c_u, w_acc, step_end, w_out):
    z = jnp.maximum(_cat((gA, gB, gC, gD)), 0.0)
    k = _gn(z, A8[...], w2[...], b2[...])
    nacc = acc[...] + w_acc * k
    outs[-1][...] = nacc
    u = nacc if step_end else y[...] + c_u * k
    s = jnp.dot(u, V[...], preferred_element_type=jnp.float32) + bias[...]
    _mm_split(s, outs[:-1], w_out)


def _ls_body(gA, gB, out):
    z = jnp.concatenate([gA[...], gB[...]], axis=1)
    lane = lax.broadcasted_iota(jnp.int32, z.shape, 1)
    valid = lane < NCLASS
    zm = jnp.where(valid, z, -jnp.inf)
    mx = jnp.max(zm, axis=1, keepdims=True)
    e = jnp.where(valid, jnp.exp(z - mx), 0.0)
    lse = jnp.log(jnp.sum(e, axis=1, keepdims=True)) + mx
    out[...] = (z - lse)[:, :NCLASS]


def _sds(shape, dtype=jnp.float32):
    return jax.ShapeDtypeStruct(shape, dtype)


_q4_in_specs = tuple(_row_spec(64) for _ in range(4))
_q4_out_specs = tuple(_row_spec(32) for _ in range(4))
_q4_out_shapes = tuple(_sds((N, 32), jnp.int32) for _ in range(4))

_mmA = _tc_call(
    _mmA_body,
    [_row_spec(NFEAT), _full_spec((NFEAT, NHID)), _full_spec((1, NHID))],
    _q4_out_specs,
    _q4_out_shapes,
)

_mmB0 = _tc_call(
    _mmB0_body,
    list(_q4_in_specs) + [_full_spec((NHID, NHID)), _full_spec((1, NHID))],
    _q4_out_specs + (_row_spec(NHID),),
    _q4_out_shapes + (_sds((N, NHID)),),
)

_mmB = _tc_call(
    _mmB_body,
    list(_q4_in_specs) + [_full_spec((1, NHID)), _full_spec((1, NHID)),
                          _full_spec((NHID, NHID)), _full_spec((NHID, NHID)),
                          _full_spec((1, NHID))],
    _q4_out_specs,
    _q4_out_shapes,
)


def _make_mmC(c_u, w_acc, step_end, nout, w_out):
    body = functools.partial(_mmC_body, c_u=c_u, w_acc=w_acc,
                             step_end=step_end, w_out=w_out)
    wp = w_out // 2
    osp = tuple(_row_spec(wp) for _ in range(nout)) + (_row_spec(NHID),)
    osh = tuple(_sds((N, wp), jnp.int32) for _ in range(nout)) + \
        (_sds((N, NHID)),)
    return _tc_call(
        body,
        list(_q4_in_specs) + [_row_spec(NHID), _row_spec(NHID),
                              _full_spec((1, NHID)), _full_spec((1, NHID)),
                              _full_spec((NHID, NHID)),
                              _full_spec((NHID, nout * w_out)),
                              _full_spec((1, nout * w_out))],
        osp,
        osh,
    )


_ls = _tc_call(
    _ls_body,
    [_row_spec(32), _row_spec(32)],
    _row_spec(NCLASS),
    _sds((N, NCLASS)),
)

_A8_NP = np.kron(np.eye(GROUPS), np.full((8, 8), 0.125)).astype(np.float32)


def kernel(x, src, tgt, Mtgt, W0, b0, gc1W, gc1b, gn1w, gn1b,
           gc2W, gc2b, gn2w, gn2b, Wl, bl):
    pad = EPAD - E
    srcT = jnp.pad(src, (0, pad)).reshape(NTILES, NCH, CHUNK)
    tgtT = jnp.pad(tgt, (0, pad)).reshape(NTILES, NCH, CHUNK)
    mT = jnp.pad(Mtgt[:, 0], (0, pad)).reshape(NTILES, NCH * CHUNK)

    _A8 = jnp.asarray(_A8_NP)
    V1, r1 = gc1W[1:], gc1W[0]
    V2, r2 = gc2W[1:], gc2W[0]

    def b1(t):
        return (gc1b + t * r1).reshape(1, NHID)

    def b2(t):
        return (gc2b + t * r2).reshape(1, NHID)

    WlP = jnp.pad(Wl, ((0, 0), (0, 64 - NCLASS)))
    blP = jnp.pad(bl, (0, 64 - NCLASS)).reshape(1, 64)
    gw1 = gn1w.reshape(1, NHID)
    gb1 = gn1b.reshape(1, NHID)
    gw2 = gn2w.reshape(1, NHID)
    gb2 = gn2b.reshape(1, NHID)

    def agg(s4):
        return _agg_main(*s4, srcT, tgtT, mT)

    s4 = _mmA(x, W0, b0.reshape(1, NHID))
    g4 = agg(s4)
    *s4, y = _mmB0(*g4, V1, b1(0.0))
    acc = y

    stage_t = [0.0, 0.25, 0.25, 0.5, 0.5, 0.75, 0.75, 1.0]
    cu = [DT / 2, DT / 2, DT, 0.0]
    wa = [DT / 6, DT / 3, DT / 3, DT / 6]
    for j in range(8):
        pos = j % 4
        g4 = agg(s4)
        s4 = _mmB(*g4, gw1, gb1, _A8, V2, b2(stage_t[j]))
        g4 = agg(s4)
        last = j == 7
        step_end = pos == 3
        mmC = _make_mmC(cu[pos], wa[pos], step_end,
                        2 if last else 4, 32 if last else 64)
        outs = mmC(*g4, y, acc,  gw2, gb2, _A8,
                   WlP if last else V1,
                   blP if last else b1(stage_t[j + 1] if not last else 0.0))
        *s4, newst = outs
        if step_end:
            y = newst
            acc = newst
        else:
            acc = newst

    gA, gB = _agg_cls(*s4, srcT, tgtT, mT)
    return _ls(gA, gB)


# speedup vs baseline: 3.3856x; 1.0047x over previous
import functools

import jax
import jax.numpy as jnp
import numpy as np
from jax import lax
from jax.experimental import pallas as pl
from jax.experimental.pallas import tpu as pltpu
from jax.experimental.pallas import tpu_sc as plsc

N = 10000
E = 160000
NFEAT = 256
NHID = 256
NCLASS = 40
GROUPS = 32
DT = 0.5

BLK = 1000
NTILES = 16
CHUNK = 128
NCH = 80
EPAD = NTILES * NCH * CHUNK
TILE_STRIDE = 632
_SCATTER = True
_SCALE = True
NBANK = 4
ACC_ROWS = NTILES * TILE_STRIDE


def _make_agg(D2, nphases, acc_rows=ACC_ROWS, accio=True):
    mesh = plsc.VectorSubcoreMesh(core_axis_name="c", subcore_axis_name="s")
    nvec = D2 // 16
    nsup = 2 * nphases
    f32 = jnp.float32

    def body(*refs):
        sups = refs[:nsup]
        srcT, tgtT, mT = refs[nsup:nsup + 3]
        outs = refs[nsup + 3:nsup + 3 + nsup]
        tail = refs[nsup + 3 + nsup:]
        src_v, tgt_v, m_v, bin_, bout, acc = tail[:6]
        gsems = tail[6:6 + NBANK]
        ssems = tail[6 + NBANK:6 + 2 * NBANK]

        cid = lax.axis_index("c")
        sid = lax.axis_index("s")

        pltpu.sync_copy(srcT.at[sid], src_v)
        pltpu.sync_copy(tgtT.at[sid], tgt_v)
        pltpu.sync_copy(mT.at[sid], m_v)

        base = sid * TILE_STRIDE
        zv = jnp.zeros((16,), f32)

        def run(sup, out, nrows_drain):
            def zrow(i, carry):
                for q in range(nvec):
                    bout[0, i, pl.ds(q * 16, 16)] = zv
                return carry
            lax.fori_loop(0, CHUNK, zrow, 0)
            if accio:
                off = 0
                while off < TILE_STRIDE:
                    nr = min(CHUNK, TILE_STRIDE - off)
                    pltpu.sync_copy(bout.at[0, pl.ds(0, nr)],
                                    acc.at[pl.ds(base + off, nr)])
                    off += nr
            plsc.subcore_barrier()

            def gstart(j, bank):
                pltpu.async_copy(sup.at[src_v.at[j]], bin_.at[bank],
                                 gsems[bank])

            def gwait(j, bank):
                pltpu.make_async_copy(sup.at[src_v.at[j]], bin_.at[bank],
                                      gsems[bank]).wait()

            def scale(j, bank):
                jbase = j * CHUNK
                hmask = jnp.full((16,), -65536, jnp.int32)

                def sgroup(g, carry):
                    v16 = m_v[pl.ds(jbase + g * 16, 16)]
                    for l in range(16):
                        m = jnp.broadcast_to(v16[l], (16,))
                        i = g * 16 + l
                        for w in range(D2 // 32):
                            v = bin_[bank, i, pl.ds(w * 16, 16)]
                            lo = lax.bitcast_convert_type(v << 16,
                                                          jnp.float32)
                            hi = lax.bitcast_convert_type(v & hmask,
                                                          jnp.float32)
                            bout[bank, i, pl.ds(w * 32, 16)] = lo * m
                            bout[bank, i, pl.ds(w * 32 + 16, 16)] = hi * m
                    return carry
                if _SCALE:
                    lax.fori_loop(0, CHUNK // 16, sgroup, 0)

            def sstart(j, bank):
                pltpu.async_copy(bout.at[bank], acc.at[tgt_v.at[j]],
                                 ssems[bank], add=True)

            def swait(j, bank):
                pltpu.make_async_copy(bout.at[bank], acc.at[tgt_v.at[j]],
                                      ssems[bank]).wait()

            for b in range(NBANK):
                gstart(b, b)

            def stepb(jj, carry):
                j = NBANK * jj
                for bank in range(NBANK):
                    jc = j + bank
                    gwait(jc, bank)

                    if _SCATTER:
                        @pl.when(jc >= NBANK)
                        def _():
                            swait(jc - NBANK, bank)
                    scale(jc, bank)
                    if _SCATTER:
                        sstart(jc, bank)

                    @pl.when(jc + NBANK < NCH)
                    def _():
                        gstart(jc + NBANK, bank)
                return carry
            lax.fori_loop(0, NCH // NBANK, stepb, 0)
            if _SCATTER:
                for b in range(NBANK):
                    swait(NCH - NBANK + b, b)

            plsc.subcore_barrier()
            if accio:
                o = 0
                while o < nrows_drain:
                    nr = min(CHUNK, nrows_drain - o)
                    pltpu.sync_copy(acc.at[pl.ds(base + o, nr)],
                                    out.at[pl.ds(base + o, nr)])
                    o += nr

        last_rows = N - (NTILES - 1) * TILE_STRIDE

        for c in range(2):
            @pl.when(cid == c)
            def _():
                for p in range(nphases):
                    i = nphases * c + p

                    @pl.when(sid < NTILES - 1)
                    def _():
                        run(sups[i], outs[i], TILE_STRIDE)

                    @pl.when(sid == NTILES - 1)
                    def _():
                        run(sups[i], outs[i], last_rows)

    sds = jax.ShapeDtypeStruct((N, D2), f32)
    return pl.kernel(
        body,
        out_type=(sds,) * nsup,
        mesh=mesh,
        scratch_types=[
            pltpu.VMEM((NCH, CHUNK), jnp.int32),
            pltpu.VMEM((NCH, CHUNK), jnp.int32),
            pltpu.VMEM((NCH * CHUNK,), f32),
            pltpu.VMEM((NBANK, CHUNK, D2 // 2), jnp.int32),
            pltpu.VMEM((NBANK, CHUNK, D2), f32),
            pltpu.VMEM_SHARED((acc_rows, D2), f32),
        ] + [pltpu.SemaphoreType.DMA] * (2 * NBANK),
        compiler_params=pltpu.CompilerParams(
            use_tc_tiling_on_sc=False,
            internal_scratch_in_bytes=256 * 1024,
        ),
    )


_agg_main = _make_agg(64, 2)
_agg_cls = _make_agg(32, 1)


def _gn(z, A8, w, b):
    m = jnp.dot(z, A8, preferred_element_type=jnp.float32)
    q = jnp.dot(z * z, A8, preferred_element_type=jnp.float32)
    inv = lax.rsqrt(q - m * m + 1e-5)
    return (z - m) * inv * w + b


def _row_spec(w):
    return pl.BlockSpec((BLK, w), lambda i: (i, 0))


def _full_spec(shape):
    nd = len(shape)
    return pl.BlockSpec(shape, lambda i: (0,) * nd)


def _tc_call(body, in_specs, out_specs, out_shapes):
    return pl.pallas_call(
        body,
        grid=(N // BLK,),
        in_specs=in_specs,
        out_specs=out_specs,
        out_shape=out_shapes,
    )


def _pack(q):
    blocks = []
    for b in range(q.shape[1] // 32):
        lo = q[:, 32 * b:32 * b + 16]
        hi = q[:, 32 * b + 16:32 * b + 32]
        lo16 = lax.bitcast_convert_type(lo.astype(jnp.bfloat16), jnp.uint16)
        hi16 = lax.bitcast_convert_type(hi.astype(jnp.bfloat16), jnp.uint16)
        w = lo16.astype(jnp.uint32) | (hi16.astype(jnp.uint32) << 16)
        blocks.append(lax.bitcast_convert_type(w, jnp.int32))
    return jnp.concatenate(blocks, axis=1) if len(blocks) > 1 else blocks[0]


def _mm_split(s, outs, w):
    for i, o in enumerate(outs):
        o[...] = _pack(s[:, i * w:(i + 1) * w])


def _cat(gs):
    return jnp.concatenate([g[...] for g in gs], axis=1)


def _mmA_body(y, V, bias, *outs):
    s = jnp.dot(y[...], V[...], preferred_element_type=jnp.float32) + bias[...]
    _mm_split(s, outs, 64)


def _mmB0_body(gA, gB, gC, gD, V, bias, *outs):
    y = jnp.maximum(_cat((gA, gB, gC, gD)), 0.0)
    outs[-1][...] = y
    s = jnp.dot(y, V[...], preferred_element_type=jnp.float32) + bias[...]
    _mm_split(s, outs[:-1], 64)


def _mmB_body(gA, gB, gC, gD, w1, b1, A8, V, bias, *outs):
    z = jnp.maximum(_cat((gA, gB, gC, gD)), 0.0)
    h = _gn(z, A8[...], w1[...], b1[...])
    s = jnp.dot(h, V[...], preferred_element_type=jnp.float32) + bias[...]
    _mm_split(s, outs, 64)


def _mmC_body(gA, gB, gC, gD, y, acc, w2, b2, A8, V, bias, *outs,
              c_u, w_acc, step_end, w_out):
    z = jnp.maximum(_cat((gA, gB, gC, gD)), 0.0)
    k = _gn(z, A8[...], w2[...], b2[...])
    nacc = acc[...] + w_acc * k
    outs[-1][...] = nacc
    u = nacc if step_end else y[...] + c_u * k
    s = jnp.dot(u, V[...], preferred_element_type=jnp.float32) + bias[...]
    _mm_split(s, outs[:-1], w_out)


def _ls_body(gA, gB, out):
    z = jnp.concatenate([gA[...], gB[...]], axis=1)
    lane = lax.broadcasted_iota(jnp.int32, z.shape, 1)
    valid = lane < NCLASS
    zm = jnp.where(valid, z, -jnp.inf)
    mx = jnp.max(zm, axis=1, keepdims=True)
    e = jnp.where(valid, jnp.exp(z - mx), 0.0)
    lse = jnp.log(jnp.sum(e, axis=1, keepdims=True)) + mx
    out[...] = (z - lse)[:, :NCLASS]


def _sds(shape, dtype=jnp.float32):
    return jax.ShapeDtypeStruct(shape, dtype)


_q4_in_specs = tuple(_row_spec(64) for _ in range(4))
_q4_out_specs = tuple(_row_spec(32) for _ in range(4))
_q4_out_shapes = tuple(_sds((N, 32), jnp.int32) for _ in range(4))

_mmA = _tc_call(
    _mmA_body,
    [_row_spec(NFEAT), _full_spec((NFEAT, NHID)), _full_spec((1, NHID))],
    _q4_out_specs,
    _q4_out_shapes,
)

_mmB0 = _tc_call(
    _mmB0_body,
    list(_q4_in_specs) + [_full_spec((NHID, NHID)), _full_spec((1, NHID))],
    _q4_out_specs + (_row_spec(NHID),),
    _q4_out_shapes + (_sds((N, NHID)),),
)

_mmB = _tc_call(
    _mmB_body,
    list(_q4_in_specs) + [_full_spec((1, NHID)), _full_spec((1, NHID)),
                          _full_spec((NHID, NHID)), _full_spec((NHID, NHID)),
                          _full_spec((1, NHID))],
    _q4_out_specs,
    _q4_out_shapes,
)


def _make_mmC(c_u, w_acc, step_end, nout, w_out):
    body = functools.partial(_mmC_body, c_u=c_u, w_acc=w_acc,
                             step_end=step_end, w_out=w_out)
    wp = w_out // 2
    osp = tuple(_row_spec(wp) for _ in range(nout)) + (_row_spec(NHID),)
    osh = tuple(_sds((N, wp), jnp.int32) for _ in range(nout)) + \
        (_sds((N, NHID)),)
    return _tc_call(
        body,
        list(_q4_in_specs) + [_row_spec(NHID), _row_spec(NHID),
                              _full_spec((1, NHID)), _full_spec((1, NHID)),
                              _full_spec((NHID, NHID)),
                              _full_spec((NHID, nout * w_out)),
                              _full_spec((1, nout * w_out))],
        osp,
        osh,
    )


_ls = _tc_call(
    _ls_body,
    [_row_spec(32), _row_spec(32)],
    _row_spec(NCLASS),
    _sds((N, NCLASS)),
)

_A8_NP = np.kron(np.eye(GROUPS), np.full((8, 8), 0.125)).astype(np.float32)


def kernel(x, src, tgt, Mtgt, W0, b0, gc1W, gc1b, gn1w, gn1b,
           gc2W, gc2b, gn2w, gn2b, Wl, bl):
    pad = EPAD - E
    srcT = jnp.pad(src, (0, pad)).reshape(NTILES, NCH, CHUNK)
    tgtT = jnp.pad(tgt, (0, pad)).reshape(NTILES, NCH, CHUNK)
    mT = jnp.pad(Mtgt[:, 0], (0, pad)).reshape(NTILES, NCH * CHUNK)

    _A8 = jnp.asarray(_A8_NP)
    V1, r1 = gc1W[1:], gc1W[0]
    V2, r2 = gc2W[1:], gc2W[0]

    def b1(t):
        return (gc1b + t * r1).reshape(1, NHID)

    def b2(t):
        return (gc2b + t * r2).reshape(1, NHID)

    WlP = jnp.pad(Wl, ((0, 0), (0, 64 - NCLASS)))
    blP = jnp.pad(bl, (0, 64 - NCLASS)).reshape(1, 64)
    gw1 = gn1w.reshape(1, NHID)
    gb1 = gn1b.reshape(1, NHID)
    gw2 = gn2w.reshape(1, NHID)
    gb2 = gn2b.reshape(1, NHID)

    def agg(s4):
        return _agg_main(*s4, srcT, tgtT, mT)

    s4 = _mmA(x, W0, b0.reshape(1, NHID))
    g4 = agg(s4)
    *s4, y = _mmB0(*g4, V1, b1(0.0))
    acc = y

    stage_t = [0.0, 0.25, 0.25, 0.5, 0.5, 0.75, 0.75, 1.0]
    cu = [DT / 2, DT / 2, DT, 0.0]
    wa = [DT / 6, DT / 3, DT / 3, DT / 6]
    for j in range(8):
        pos = j % 4
        g4 = agg(s4)
        s4 = _mmB(*g4, gw1, gb1, _A8, V2, b2(stage_t[j]))
        g4 = agg(s4)
        last = j == 7
        step_end = pos == 3
        mmC = _make_mmC(cu[pos], wa[pos], step_end,
                        2 if last else 4, 32 if last else 64)
        outs = mmC(*g4, y, acc,  gw2, gb2, _A8,
                   WlP if last else V1,
                   blP if last else b1(stage_t[j + 1] if not last else 0.0))
        *s4, newst = outs
        if step_end:
            y = newst
            acc = newst
        else:
            acc = newst

    gA, gB = _agg_cls(*s4, srcT, tgtT, mT)
    return _ls(gA, gB)


# speedup vs baseline: 3.3966x; 1.0033x over previous
import functools

import jax
import jax.numpy as jnp
import numpy as np
from jax import lax
from jax.experimental import pallas as pl
from jax.experimental.pallas import tpu as pltpu
from jax.experimental.pallas import tpu_sc as plsc

N = 10000
E = 160000
NFEAT = 256
NHID = 256
NCLASS = 40
GROUPS = 32
DT = 0.5

BLK = 1000
NTILES = 16
CHUNK = 128
NCH = 80
EPAD = NTILES * NCH * CHUNK
TILE_STRIDE = 632
NBANK = 4
ACC_ROWS = NTILES * TILE_STRIDE


def _make_agg(D2, nphases, acc_rows=ACC_ROWS):
    mesh = plsc.VectorSubcoreMesh(core_axis_name="c", subcore_axis_name="s")
    nvec = D2 // 16
    nsup = 2 * nphases
    f32 = jnp.float32

    def body(*refs):
        sups = refs[:nsup]
        srcT, tgtT, mT = refs[nsup:nsup + 3]
        outs = refs[nsup + 3:nsup + 3 + nsup]
        tail = refs[nsup + 3 + nsup:]
        src_v, tgt_v, m_v, bin_, bout, acc = tail[:6]
        gsems = tail[6:6 + NBANK]
        ssems = tail[6 + NBANK:6 + 2 * NBANK]

        cid = lax.axis_index("c")
        sid = lax.axis_index("s")

        pltpu.sync_copy(srcT.at[sid], src_v)
        pltpu.sync_copy(tgtT.at[sid], tgt_v)
        pltpu.sync_copy(mT.at[sid], m_v)

        base = sid * TILE_STRIDE
        zv = jnp.zeros((16,), f32)

        def run(sup, out, nrows_drain):
            def zrow(i, carry):
                for q in range(nvec):
                    bout[0, i, pl.ds(q * 16, 16)] = zv
                return carry
            lax.fori_loop(0, CHUNK, zrow, 0)
            off = 0
            while off < TILE_STRIDE:
                nr = min(CHUNK, TILE_STRIDE - off)
                pltpu.sync_copy(bout.at[0, pl.ds(0, nr)],
                                acc.at[pl.ds(base + off, nr)])
                off += nr
            plsc.subcore_barrier()

            def gstart(j, bank):
                pltpu.async_copy(sup.at[src_v.at[j]], bin_.at[bank],
                                 gsems[bank])

            def gwait(j, bank):
                pltpu.make_async_copy(sup.at[src_v.at[j]], bin_.at[bank],
                                      gsems[bank]).wait()

            def scale(j, bank):
                jbase = j * CHUNK
                hmask = jnp.full((16,), -65536, jnp.int32)

                def sgroup(g, carry):
                    v16 = m_v[pl.ds(jbase + g * 16, 16)]
                    for l in range(16):
                        m = jnp.broadcast_to(v16[l], (16,))
                        i = g * 16 + l
                        for w in range(D2 // 32):
                            v = bin_[bank, i, pl.ds(w * 16, 16)]
                            lo = lax.bitcast_convert_type(v << 16,
                                                          jnp.float32)
                            hi = lax.bitcast_convert_type(v & hmask,
                                                          jnp.float32)
                            bout[bank, i, pl.ds(w * 32, 16)] = lo * m
                            bout[bank, i, pl.ds(w * 32 + 16, 16)] = hi * m
                    return carry
                lax.fori_loop(0, CHUNK // 16, sgroup, 0)

            def sstart(j, bank):
                pltpu.async_copy(bout.at[bank], acc.at[tgt_v.at[j]],
                                 ssems[bank], add=True)

            def swait(j, bank):
                pltpu.make_async_copy(bout.at[bank], acc.at[tgt_v.at[j]],
                                      ssems[bank]).wait()

            for b in range(NBANK):
                gstart(b, b)

            def stepb(jj, carry):
                j = NBANK * jj
                for bank in range(NBANK):
                    jc = j + bank
                    gwait(jc, bank)

                    @pl.when(jc >= NBANK)
                    def _():
                        swait(jc - NBANK, bank)
                    scale(jc, bank)
                    sstart(jc, bank)

                    @pl.when(jc + NBANK < NCH)
                    def _():
                        gstart(jc + NBANK, bank)
                return carry
            lax.fori_loop(0, NCH // NBANK, stepb, 0)
            for b in range(NBANK):
                swait(NCH - NBANK + b, b)

            plsc.subcore_barrier()
            o = 0
            while o < nrows_drain:
                nr = min(CHUNK, nrows_drain - o)
                pltpu.sync_copy(acc.at[pl.ds(base + o, nr)],
                                out.at[pl.ds(base + o, nr)])
                o += nr

        last_rows = N - (NTILES - 1) * TILE_STRIDE

        for c in range(2):
            @pl.when(cid == c)
            def _():
                for p in range(nphases):
                    i = nphases * c + p

                    @pl.when(sid < NTILES - 1)
                    def _():
                        run(sups[i], outs[i], TILE_STRIDE)

                    @pl.when(sid == NTILES - 1)
                    def _():
                        run(sups[i], outs[i], last_rows)

    sds = jax.ShapeDtypeStruct((N, D2), f32)
    return pl.kernel(
        body,
        out_type=(sds,) * nsup,
        mesh=mesh,
        scratch_types=[
            pltpu.VMEM((NCH, CHUNK), jnp.int32),
            pltpu.VMEM((NCH, CHUNK), jnp.int32),
            pltpu.VMEM((NCH * CHUNK,), f32),
            pltpu.VMEM((NBANK, CHUNK, D2 // 2), jnp.int32),
            pltpu.VMEM((NBANK, CHUNK, D2), f32),
            pltpu.VMEM_SHARED((acc_rows, D2), f32),
        ] + [pltpu.SemaphoreType.DMA] * (2 * NBANK),
        compiler_params=pltpu.CompilerParams(
            use_tc_tiling_on_sc=False,
            internal_scratch_in_bytes=256 * 1024,
        ),
    )


_agg_main = _make_agg(64, 2)
_agg_cls = _make_agg(32, 1)


def _gn(z, A8, w, b):
    m = jnp.dot(z, A8, preferred_element_type=jnp.float32)
    q = jnp.dot(z * z, A8, preferred_element_type=jnp.float32)
    inv = lax.rsqrt(q - m * m + 1e-5)
    return (z - m) * inv * w + b


def _row_spec(w):
    return pl.BlockSpec((BLK, w), lambda i: (i, 0))


def _full_spec(shape):
    nd = len(shape)
    return pl.BlockSpec(shape, lambda i: (0,) * nd)


def _tc_call(body, in_specs, out_specs, out_shapes):
    return pl.pallas_call(
        body,
        grid=(N // BLK,),
        in_specs=in_specs,
        out_specs=out_specs,
        out_shape=out_shapes,
    )


def _pack(q):
    blocks = []
    for b in range(q.shape[1] // 32):
        lo = q[:, 32 * b:32 * b + 16]
        hi = q[:, 32 * b + 16:32 * b + 32]
        lo16 = lax.bitcast_convert_type(lo.astype(jnp.bfloat16), jnp.uint16)
        hi16 = lax.bitcast_convert_type(hi.astype(jnp.bfloat16), jnp.uint16)
        w = lo16.astype(jnp.uint32) | (hi16.astype(jnp.uint32) << 16)
        blocks.append(lax.bitcast_convert_type(w, jnp.int32))
    return jnp.concatenate(blocks, axis=1) if len(blocks) > 1 else blocks[0]


def _mm_split(s, outs, w):
    for i, o in enumerate(outs):
        o[...] = _pack(s[:, i * w:(i + 1) * w])


def _cat(gs):
    return jnp.concatenate([g[...] for g in gs], axis=1)


def _mmA_body(y, V, bias, *outs):
    s = jnp.dot(y[...], V[...], preferred_element_type=jnp.float32) + bias[...]
    _mm_split(s, outs, 64)


def _mmB0_body(gA, gB, gC, gD, V, bias, *outs):
    y = jnp.maximum(_cat((gA, gB, gC, gD)), 0.0)
    outs[-1][...] = y
    s = jnp.dot(y, V[...], preferred_element_type=jnp.float32) + bias[...]
    _mm_split(s, outs[:-1], 64)


def _mmB_body(gA, gB, gC, gD, w1, b1, A8, V, bias, *outs):
    z = jnp.maximum(_cat((gA, gB, gC, gD)), 0.0)
    h = _gn(z, A8[...], w1[...], b1[...])
    s = jnp.dot(h, V[...], preferred_element_type=jnp.float32) + bias[...]
    _mm_split(s, outs, 64)


def _mmC_body(gA, gB, gC, gD, y, acc, w2, b2, A8, V, bias, *outs,
              c_u, w_acc, step_end, w_out):
    z = jnp.maximum(_cat((gA, gB, gC, gD)), 0.0)
    k = _gn(z, A8[...], w2[...], b2[...])
    nacc = acc[...] + w_acc * k
    outs[-1][...] = nacc
    u = nacc if step_end else y[...] + c_u * k
    s = jnp.dot(u, V[...], preferred_element_type=jnp.float32) + bias[...]
    _mm_split(s, outs[:-1], w_out)


def _ls_body(gA, gB, out):
    z = jnp.concatenate([gA[...], gB[...]], axis=1)
    lane = lax.broadcasted_iota(jnp.int32, z.shape, 1)
    valid = lane < NCLASS
    zm = jnp.where(valid, z, -jnp.inf)
    mx = jnp.max(zm, axis=1, keepdims=True)
    e = jnp.where(valid, jnp.exp(z - mx), 0.0)
    lse = jnp.log(jnp.sum(e, axis=1, keepdims=True)) + mx
    out[...] = (z - lse)[:, :NCLASS]


def _sds(shape, dtype=jnp.float32):
    return jax.ShapeDtypeStruct(shape, dtype)


_q4_in_specs = tuple(_row_spec(64) for _ in range(4))
_q4_out_specs = tuple(_row_spec(32) for _ in range(4))
_q4_out_shapes = tuple(_sds((N, 32), jnp.int32) for _ in range(4))

_mmA = _tc_call(
    _mmA_body,
    [_row_spec(NFEAT), _full_spec((NFEAT, NHID)), _full_spec((1, NHID))],
    _q4_out_specs,
    _q4_out_shapes,
)

_mmB0 = _tc_call(
    _mmB0_body,
    list(_q4_in_specs) + [_full_spec((NHID, NHID)), _full_spec((1, NHID))],
    _q4_out_specs + (_row_spec(NHID),),
    _q4_out_shapes + (_sds((N, NHID)),),
)

_mmB = _tc_call(
    _mmB_body,
    list(_q4_in_specs) + [_full_spec((1, NHID)), _full_spec((1, NHID)),
                          _full_spec((NHID, NHID)), _full_spec((NHID, NHID)),
                          _full_spec((1, NHID))],
    _q4_out_specs,
    _q4_out_shapes,
)


def _make_mmC(c_u, w_acc, step_end, nout, w_out):
    body = functools.partial(_mmC_body, c_u=c_u, w_acc=w_acc,
                             step_end=step_end, w_out=w_out)
    wp = w_out // 2
    osp = tuple(_row_spec(wp) for _ in range(nout)) + (_row_spec(NHID),)
    osh = tuple(_sds((N, wp), jnp.int32) for _ in range(nout)) + \
        (_sds((N, NHID)),)
    return _tc_call(
        body,
        list(_q4_in_specs) + [_row_spec(NHID), _row_spec(NHID),
                              _full_spec((1, NHID)), _full_spec((1, NHID)),
                              _full_spec((NHID, NHID)),
                              _full_spec((NHID, nout * w_out)),
                              _full_spec((1, nout * w_out))],
        osp,
        osh,
    )


_ls = _tc_call(
    _ls_body,
    [_row_spec(32), _row_spec(32)],
    _row_spec(NCLASS),
    _sds((N, NCLASS)),
)

_A8_NP = np.kron(np.eye(GROUPS), np.full((8, 8), 0.125)).astype(np.float32)


def kernel(x, src, tgt, Mtgt, W0, b0, gc1W, gc1b, gn1w, gn1b,
           gc2W, gc2b, gn2w, gn2b, Wl, bl):
    pad = EPAD - E
    srcT = jnp.pad(src, (0, pad)).reshape(NTILES, NCH, CHUNK)
    tgtT = jnp.pad(tgt, (0, pad)).reshape(NTILES, NCH, CHUNK)
    mT = jnp.pad(Mtgt[:, 0], (0, pad)).reshape(NTILES, NCH * CHUNK)

    _A8 = jnp.asarray(_A8_NP)
    V1, r1 = gc1W[1:], gc1W[0]
    V2, r2 = gc2W[1:], gc2W[0]

    def b1(t):
        return (gc1b + t * r1).reshape(1, NHID)

    def b2(t):
        return (gc2b + t * r2).reshape(1, NHID)

    WlP = jnp.pad(Wl, ((0, 0), (0, 64 - NCLASS)))
    blP = jnp.pad(bl, (0, 64 - NCLASS)).reshape(1, 64)
    gw1 = gn1w.reshape(1, NHID)
    gb1 = gn1b.reshape(1, NHID)
    gw2 = gn2w.reshape(1, NHID)
    gb2 = gn2b.reshape(1, NHID)

    def agg(s4):
        return _agg_main(*s4, srcT, tgtT, mT)

    s4 = _mmA(x, W0, b0.reshape(1, NHID))
    g4 = agg(s4)
    *s4, y = _mmB0(*g4, V1, b1(0.0))
    acc = y

    stage_t = [0.0, 0.25, 0.25, 0.5, 0.5, 0.75, 0.75, 1.0]
    cu = [DT / 2, DT / 2, DT, 0.0]
    wa = [DT / 6, DT / 3, DT / 3, DT / 6]
    for j in range(8):
        pos = j % 4
        g4 = agg(s4)
        s4 = _mmB(*g4, gw1, gb1, _A8, V2, b2(stage_t[j]))
        g4 = agg(s4)
        last = j == 7
        step_end = pos == 3
        mmC = _make_mmC(cu[pos], wa[pos], step_end,
                        2 if last else 4, 32 if last else 64)
        outs = mmC(*g4, y, acc,  gw2, gb2, _A8,
                   WlP if last else V1,
                   blP if last else b1(stage_t[j + 1] if not last else 0.0))
        *s4, newst = outs
        if step_end:
            y = newst
            acc = newst
        else:
            acc = newst

    gA, gB = _agg_cls(*s4, srcT, tgtT, mT)
    return _ls(gA, gB)
